# Initial kernel scaffold; baseline (speedup 1.0000x reference)
import jax
import jax.numpy as jnp
from jax.experimental import pallas as pl


def kernel(x, f, W1, b1, g1, be1, W2, b2, g2, be2):
    raise NotImplementedError("write your pallas kernel here")


# speedup vs baseline: 12.2701x; 12.2701x over previous
import functools

import jax
import jax.numpy as jnp
import numpy as np
from jax import lax
from jax.experimental import pallas as pl
from jax.experimental.pallas import tpu as pltpu
from jax.experimental.pallas import tpu_sc as plsc

B, N, S, K = 8, 4096, 512, 32
DIN, DMID, DOUT = 128, 128, 256
R2 = np.float32(0.2 ** 2)
INV_SQRT = np.float32(1.0) / np.sqrt(np.float32(1.0 + 1e-5))
NCHUNK = N // 128


def _embed_body(f_ref, w1_ref, b1_ref, w2_ref, b2_ref, o_ref):
    xb = f_ref[...].astype(jnp.bfloat16)
    h = jax.lax.dot_general(xb, w1_ref[...].astype(jnp.bfloat16),
                            (((1,), (0,)), ((), ())),
                            preferred_element_type=jnp.float32)
    h = h + b1_ref[...]
    h = jnp.maximum(h, 0.0)
    h2 = jax.lax.dot_general(h.astype(jnp.bfloat16),
                             w2_ref[...].astype(jnp.bfloat16),
                             (((1,), (0,)), ((), ())),
                             preferred_element_type=jnp.float32)
    o_ref[...] = jnp.maximum(h2 + b2_ref[...], 0.0)


def _embed(f2d, w1s, b1s, w2s, b2s):
    blk = 1024
    return pl.pallas_call(
        _embed_body,
        grid=(B * N // blk,),
        in_specs=[pl.BlockSpec((blk, DIN), lambda i: (i, 0)),
                  pl.BlockSpec((DIN, DMID), lambda i: (0, 0)),
                  pl.BlockSpec((1, DMID), lambda i: (0, 0)),
                  pl.BlockSpec((DMID, DOUT), lambda i: (0, 0)),
                  pl.BlockSpec((1, DOUT), lambda i: (0, 0))],
        out_specs=pl.BlockSpec((blk, DOUT), lambda i: (i, 0)),
        out_shape=jax.ShapeDtypeStruct((B * N, DOUT), jnp.float32),
    )(f2d, w1s, b1s, w2s, b2s)


def _fps_body(xt_ref, xs_ref, gidx_ref, dmin_ref):
    xt = xt_ref[...]
    lane = lax.broadcasted_iota(jnp.int32, (B, N), 1)
    slane = lax.broadcasted_iota(jnp.int32, (B, S), 1)
    slane3 = lax.broadcasted_iota(jnp.int32, (3, B, S), 2)
    boff = lax.broadcasted_iota(jnp.int32, (B, 1), 0) * N
    dmin_ref[...] = jnp.full((B, N), 1e10, jnp.float32)

    def step(i, far):
        gidx_ref[...] = jnp.where(slane == i, far + boff, gidx_ref[...])
        oh = lane == far
        sel = jnp.where(oh[None], xt, 0.0)
        c = jnp.sum(sel, axis=2, keepdims=True)
        xs_ref[...] = jnp.where(slane3 == i, c, xs_ref[...])
        d = xt - c
        d = d * d
        dist = (d[0] + d[1]) + d[2]
        dm = jnp.minimum(dmin_ref[...], dist)
        dmin_ref[...] = dm
        m = jnp.max(dm, axis=1, keepdims=True)
        far_new = jnp.min(jnp.where(dm == m, lane, jnp.int32(N)), axis=1,
                          keepdims=True)
        return far_new

    lax.fori_loop(0, S, step, jnp.zeros((B, 1), jnp.int32))


def _fps(xt):
    return pl.pallas_call(
        _fps_body,
        in_specs=[pl.BlockSpec((3, B, N), lambda: (0, 0, 0))],
        out_specs=[pl.BlockSpec((3, B, S), lambda: (0, 0, 0)),
                   pl.BlockSpec((B, S), lambda: (0, 0))],
        out_shape=[jax.ShapeDtypeStruct((3, B, S), jnp.float32),
                   jax.ShapeDtypeStruct((B, S), jnp.int32)],
        scratch_shapes=[pltpu.VMEM((B, N), jnp.float32)],
    )(xt)


def _ballq_body(xs_ref, xt_ref, o_ref, base_scr, acc_scr):
    b = pl.program_id(0)
    j = pl.program_id(1)

    @pl.when(j == 0)
    def _():
        base_scr[...] = jnp.zeros((S, 1), jnp.float32)
        acc_scr[...] = jnp.zeros((K, S), jnp.float32)

    a = xs_ref[0]
    bt = xt_ref[0]
    e = jax.lax.dot_general(a.astype(jnp.bfloat16), bt.astype(jnp.bfloat16),
                            (((1,), (0,)), ((), ())),
                            preferred_element_type=jnp.float32)
    s2 = (a[:, 0:1] * a[:, 0:1] + a[:, 1:2] * a[:, 1:2]) + a[:, 2:3] * a[:, 2:3]
    n2 = (bt[0:1] * bt[0:1] + bt[1:2] * bt[1:2]) + bt[2:3] * bt[2:3]
    d = (e * (-2.0) + s2) + n2
    mask = jnp.logical_not(d > R2).astype(jnp.bfloat16)

    tri = (lax.broadcasted_iota(jnp.int32, (128, 128), 0)
           <= lax.broadcasted_iota(jnp.int32, (128, 128), 1)).astype(jnp.bfloat16)
    lc = jax.lax.dot_general(mask, tri, (((1,), (0,)), ((), ())),
                             preferred_element_type=jnp.float32)
    lc = lc + base_scr[...]
    base_scr[...] = lc[:, 127:128]
    u = jnp.minimum(lc, 34.0)

    cols = []
    for k in range(K):
        le = jnp.where(u <= np.float32(k), 1.0, 0.0)
        cols.append(jnp.sum(le, axis=1))
    upd = jnp.stack(cols, axis=0)
    acc_scr[...] += upd

    @pl.when(j == NCHUNK - 1)
    def _():
        pos = acc_scr[...]
        pos = jnp.where(pos == np.float32(N), pos[0:1, :], pos)
        o_ref[0] = pos.astype(jnp.int32) + b * N


def _ballq(xs, xt):
    return pl.pallas_call(
        _ballq_body,
        grid=(B, NCHUNK),
        in_specs=[pl.BlockSpec((1, S, 3), lambda b, j: (b, 0, 0)),
                  pl.BlockSpec((1, 3, 128), lambda b, j: (b, 0, j))],
        out_specs=pl.BlockSpec((1, K, S), lambda b, j: (b, 0, 0)),
        out_shape=jax.ShapeDtypeStruct((B, K, S), jnp.int32),
        scratch_shapes=[pltpu.VMEM((S, 1), jnp.float32),
                        pltpu.VMEM((K, S), jnp.float32)],
    )(xs, xt)


try:
    _info = plsc.get_sparse_core_info()
    _NC, _NS = _info.num_cores, _info.num_subcores
except Exception:
    _NC, _NS = 2, 16
_NW = _NC * _NS
_FG_PW = (B * S * K) // _NW
_FS_PW = (B * S) // _NW
_CH = 128
_NIT = _FG_PW // _CH


def _sc_gather(emb, xplanes, gidx, fgidx):
    mesh = plsc.VectorSubcoreMesh(core_axis_name="c", subcore_axis_name="s")

    @functools.partial(
        pl.kernel,
        out_type=[jax.ShapeDtypeStruct((B * S * K, DOUT), jnp.float32),
                  jax.ShapeDtypeStruct((3 * B * S * K,), jnp.float32),
                  jax.ShapeDtypeStruct((B * S, DOUT), jnp.float32)],
        mesh=mesh,
        compiler_params=pltpu.CompilerParams(needs_layout_passes=False),
        scratch_types=[pltpu.VMEM((_FG_PW,), jnp.int32),
                       pltpu.VMEM((_CH, DOUT), jnp.float32),
                       pltpu.VMEM((N,), jnp.float32),
                       pltpu.VMEM((N,), jnp.float32),
                       pltpu.VMEM((N,), jnp.float32),
                       pltpu.VMEM((_FG_PW,), jnp.float32),
                       pltpu.VMEM((_FG_PW,), jnp.float32),
                       pltpu.VMEM((_FG_PW,), jnp.float32),
                       pltpu.VMEM((_FS_PW,), jnp.int32),
                       pltpu.VMEM((_FS_PW, DOUT), jnp.float32),
                       pltpu.SemaphoreType.DMA],
    )
    def k(emb_hbm, xpl_hbm, gidx_hbm, fgidx_hbm, fg_hbm, xg_hbm, fs_hbm,
          idx_v, rows_v, xt0, xt1, xt2, xo0, xo1, xo2, fidx_v, frows_v, sem):
        wid = lax.axis_index("s") * _NC + lax.axis_index("c")
        base = wid * _FG_PW
        b = wid // (_NW // B)
        pltpu.sync_copy(gidx_hbm.at[pl.ds(base, _FG_PW)], idx_v)

        xtabs = (xt0, xt1, xt2)
        xouts = (xo0, xo1, xo2)
        for p in range(3):
            pltpu.sync_copy(xpl_hbm.at[pl.ds((p * B + b) * N, N)], xtabs[p])

        def fg_chunk(j, _):
            off = j * _CH
            pltpu.async_copy(emb_hbm.at[idx_v.at[pl.ds(off, _CH)]], rows_v,
                             sem).wait()
            pltpu.sync_copy(rows_v, fg_hbm.at[pl.ds(base + off, _CH)])
            return 0

        lax.fori_loop(0, _NIT, fg_chunk, 0)

        boff = b * N

        def xg_step(i, _):
            ids = idx_v[pl.ds(i * 16, 16)] - boff
            for p in range(3):
                xouts[p][pl.ds(i * 16, 16)] = plsc.load_gather(xtabs[p], [ids])
            return 0

        lax.fori_loop(0, _FG_PW // 16, xg_step, 0)
        for p in range(3):
            pltpu.sync_copy(xouts[p],
                            xg_hbm.at[pl.ds(p * (B * S * K) + base, _FG_PW)])

        fbase = wid * _FS_PW
        pltpu.sync_copy(fgidx_hbm.at[pl.ds(fbase, _FS_PW)], fidx_v)
        pltpu.async_copy(emb_hbm.at[fidx_v], frows_v, sem).wait()
        pltpu.sync_copy(frows_v, fs_hbm.at[pl.ds(fbase, _FS_PW)])

    return k(emb, xplanes, gidx, fgidx)


def kernel(x, f, W1, b1, g1, be1, W2, b2, g2, be2):
    xt3 = jnp.transpose(x, (2, 0, 1))
    xt = jnp.transpose(x, (0, 2, 1))

    s1 = g1 * INV_SQRT
    s2 = g2 * INV_SQRT
    w1s = (W1 * s1[:, None]).T
    b1s = (b1 * s1 + be1)[None, :]
    w2s = (W2 * s2[:, None]).T
    b2s = (b2 * s2 + be2)[None, :]

    emb = _embed(f.reshape(B * N, DIN), w1s, b1s, w2s, b2s)

    xs_t, fps_gidx = _fps(xt3)
    xs = jnp.transpose(xs_t, (1, 2, 0))

    gidx = jnp.transpose(_ballq(xs, xt), (0, 2, 1))

    fg_flat, xg_pl, fs_flat = _sc_gather(
        emb, xt3.reshape(3 * B * N), gidx.reshape(B * S * K),
        fps_gidx.reshape(B * S))

    xg = jnp.transpose(xg_pl.reshape(3, B * S * K), (1, 0)).reshape(B, S, K, 3)
    fg = fg_flat.reshape(B, S, K, DOUT)
    fs = fs_flat.reshape(B, S, DOUT)
    return (xg, fg, xs, fs)


# speedup vs baseline: 17.9208x; 1.4605x over previous
import functools

import jax
import jax.numpy as jnp
import numpy as np
from jax import lax
from jax.experimental import pallas as pl
from jax.experimental.pallas import tpu as pltpu
from jax.experimental.pallas import tpu_sc as plsc

B, N, S, K = 8, 4096, 512, 32
DIN, DMID, DOUT = 128, 128, 256
R2 = np.float32(0.2 ** 2)
INV_SQRT = np.float32(1.0) / np.sqrt(np.float32(1.0 + 1e-5))
NCHUNK = N // 128


def _embed_body(f_ref, w1_ref, b1_ref, w2_ref, b2_ref, o_ref):
    xb = f_ref[...].astype(jnp.bfloat16)
    h = jax.lax.dot_general(xb, w1_ref[...].astype(jnp.bfloat16),
                            (((1,), (0,)), ((), ())),
                            preferred_element_type=jnp.float32)
    h = h + b1_ref[...]
    h = jnp.maximum(h, 0.0)
    h2 = jax.lax.dot_general(h.astype(jnp.bfloat16),
                             w2_ref[...].astype(jnp.bfloat16),
                             (((1,), (0,)), ((), ())),
                             preferred_element_type=jnp.float32)
    o_ref[...] = jnp.maximum(h2 + b2_ref[...], 0.0)


def _embed(f2d, w1s, b1s, w2s, b2s):
    blk = 1024
    return pl.pallas_call(
        _embed_body,
        grid=(B * N // blk,),
        in_specs=[pl.BlockSpec((blk, DIN), lambda i: (i, 0)),
                  pl.BlockSpec((DIN, DMID), lambda i: (0, 0)),
                  pl.BlockSpec((1, DMID), lambda i: (0, 0)),
                  pl.BlockSpec((DMID, DOUT), lambda i: (0, 0)),
                  pl.BlockSpec((1, DOUT), lambda i: (0, 0))],
        out_specs=pl.BlockSpec((blk, DOUT), lambda i: (i, 0)),
        out_shape=jax.ShapeDtypeStruct((B * N, DOUT), jnp.float32),
    )(f2d, w1s, b1s, w2s, b2s)


def _fps_body(xt_ref, xs_ref, gidx_ref, dmin_ref):
    xt = xt_ref[...]
    lane = lax.broadcasted_iota(jnp.int32, (B, N), 1)
    slane = lax.broadcasted_iota(jnp.int32, (B, S), 1)
    slane3 = lax.broadcasted_iota(jnp.int32, (3, B, S), 2)
    boff = lax.broadcasted_iota(jnp.int32, (B, 1), 0) * N
    dmin_ref[...] = jnp.full((B, N), 1e10, jnp.float32)

    def step(i, far):
        gidx_ref[...] = jnp.where(slane == i, far + boff, gidx_ref[...])
        oh = lane == far
        sel = jnp.where(oh[None], xt, 0.0)
        c = jnp.sum(sel, axis=2, keepdims=True)
        xs_ref[...] = jnp.where(slane3 == i, c, xs_ref[...])
        d = xt - c
        d = d * d
        dist = (d[0] + d[1]) + d[2]
        dm = jnp.minimum(dmin_ref[...], dist)
        dmin_ref[...] = dm
        m = jnp.max(dm, axis=1, keepdims=True)
        far_new = jnp.min(jnp.where(dm == m, lane, jnp.int32(N)), axis=1,
                          keepdims=True)
        return far_new

    lax.fori_loop(0, S, step, jnp.zeros((B, 1), jnp.int32))


def _fps(xt):
    return pl.pallas_call(
        _fps_body,
        in_specs=[pl.BlockSpec((3, B, N), lambda: (0, 0, 0))],
        out_specs=[pl.BlockSpec((3, B, S), lambda: (0, 0, 0)),
                   pl.BlockSpec((B, S), lambda: (0, 0))],
        out_shape=[jax.ShapeDtypeStruct((3, B, S), jnp.float32),
                   jax.ShapeDtypeStruct((B, S), jnp.int32)],
        scratch_shapes=[pltpu.VMEM((B, N), jnp.float32)],
    )(xt)


def _ballq_body(xc_ref, at_ref, o_ref, base_scr, acc_scr):
    b = pl.program_id(0)
    j = pl.program_id(1)

    @pl.when(j == 0)
    def _():
        base_scr[...] = jnp.zeros((1, S), jnp.float32)
        acc_scr[...] = jnp.zeros((K, S), jnp.float32)

    xc = xc_ref[0]
    at = at_ref[0]
    eT = jax.lax.dot_general(xc.astype(jnp.bfloat16), at.astype(jnp.bfloat16),
                             (((1,), (0,)), ((), ())),
                             preferred_element_type=jnp.float32)
    s2T = (at[0:1] * at[0:1] + at[1:2] * at[1:2]) + at[2:3] * at[2:3]
    n2T = (xc[:, 0:1] * xc[:, 0:1] + xc[:, 1:2] * xc[:, 1:2]) + xc[:, 2:3] * xc[:, 2:3]
    dT = (eT * (-2.0) + s2T) + n2T
    maskT = jnp.logical_not(dT > R2).astype(jnp.bfloat16)

    tril = (lax.broadcasted_iota(jnp.int32, (128, 128), 1)
            <= lax.broadcasted_iota(jnp.int32, (128, 128), 0)).astype(jnp.bfloat16)
    lc = jax.lax.dot_general(tril, maskT, (((1,), (0,)), ((), ())),
                             preferred_element_type=jnp.float32)
    lc = lc + base_scr[...]
    base_scr[...] = lc[127:128, :]
    u = jnp.minimum(lc, 34.0)

    cols = []
    for k in range(K):
        le = jnp.where(u <= np.float32(k), 1.0, 0.0)
        cols.append(jnp.sum(le, axis=0, keepdims=True))
    acc_scr[...] += jnp.concatenate(cols, axis=0)

    @pl.when(j == NCHUNK - 1)
    def _():
        pos = acc_scr[...]
        pos = jnp.where(pos == np.float32(N), pos[0:1, :], pos)
        o_ref[0] = pos.astype(jnp.int32) + b * N


def _ballq(x, xst):
    return pl.pallas_call(
        _ballq_body,
        grid=(B, NCHUNK),
        in_specs=[pl.BlockSpec((1, 128, 3), lambda b, j: (b, j, 0)),
                  pl.BlockSpec((1, 3, S), lambda b, j: (b, 0, 0))],
        out_specs=pl.BlockSpec((1, K, S), lambda b, j: (b, 0, 0)),
        out_shape=jax.ShapeDtypeStruct((B, K, S), jnp.int32),
        scratch_shapes=[pltpu.VMEM((1, S), jnp.float32),
                        pltpu.VMEM((K, S), jnp.float32)],
    )(x, xst)


try:
    _info = plsc.get_sparse_core_info()
    _NC, _NS = _info.num_cores, _info.num_subcores
except Exception:
    _NC, _NS = 2, 16
_NW = _NC * _NS
_FG_PW = (B * S * K) // _NW
_FS_PW = (B * S) // _NW
_CH = 128
_NIT = _FG_PW // _CH


def _sc_gather(emb, xplanes, gidx, fgidx):
    mesh = plsc.VectorSubcoreMesh(core_axis_name="c", subcore_axis_name="s")

    @functools.partial(
        pl.kernel,
        out_type=[jax.ShapeDtypeStruct((B * S * K, DOUT), jnp.float32),
                  jax.ShapeDtypeStruct((3 * B * S * K,), jnp.float32),
                  jax.ShapeDtypeStruct((B * S, DOUT), jnp.float32)],
        mesh=mesh,
        compiler_params=pltpu.CompilerParams(needs_layout_passes=False),
        scratch_types=[pltpu.VMEM((_FG_PW,), jnp.int32),
                       pltpu.VMEM((_CH, DOUT), jnp.float32),
                       pltpu.VMEM((N,), jnp.float32),
                       pltpu.VMEM((N,), jnp.float32),
                       pltpu.VMEM((N,), jnp.float32),
                       pltpu.VMEM((_FG_PW,), jnp.float32),
                       pltpu.VMEM((_FG_PW,), jnp.float32),
                       pltpu.VMEM((_FG_PW,), jnp.float32),
                       pltpu.VMEM((_FS_PW,), jnp.int32),
                       pltpu.VMEM((_FS_PW, DOUT), jnp.float32),
                       pltpu.SemaphoreType.DMA],
    )
    def k(emb_hbm, xpl_hbm, gidx_hbm, fgidx_hbm, fg_hbm, xg_hbm, fs_hbm,
          idx_v, rows_v, xt0, xt1, xt2, xo0, xo1, xo2, fidx_v, frows_v, sem):
        wid = lax.axis_index("s") * _NC + lax.axis_index("c")
        base = wid * _FG_PW
        b = wid // (_NW // B)
        pltpu.sync_copy(gidx_hbm.at[pl.ds(base, _FG_PW)], idx_v)

        xtabs = (xt0, xt1, xt2)
        xouts = (xo0, xo1, xo2)
        for p in range(3):
            pltpu.sync_copy(xpl_hbm.at[pl.ds((p * B + b) * N, N)], xtabs[p])

        def fg_chunk(j, _):
            off = j * _CH
            pltpu.async_copy(emb_hbm.at[idx_v.at[pl.ds(off, _CH)]], rows_v,
                             sem).wait()
            pltpu.sync_copy(rows_v, fg_hbm.at[pl.ds(base + off, _CH)])
            return 0

        lax.fori_loop(0, _NIT, fg_chunk, 0)

        boff = b * N

        def xg_step(i, _):
            ids = idx_v[pl.ds(i * 16, 16)] - boff
            for p in range(3):
                xouts[p][pl.ds(i * 16, 16)] = plsc.load_gather(xtabs[p], [ids])
            return 0

        lax.fori_loop(0, _FG_PW // 16, xg_step, 0)
        for p in range(3):
            pltpu.sync_copy(xouts[p],
                            xg_hbm.at[pl.ds(p * (B * S * K) + base, _FG_PW)])

        fbase = wid * _FS_PW
        pltpu.sync_copy(fgidx_hbm.at[pl.ds(fbase, _FS_PW)], fidx_v)
        pltpu.async_copy(emb_hbm.at[fidx_v], frows_v, sem).wait()
        pltpu.sync_copy(frows_v, fs_hbm.at[pl.ds(fbase, _FS_PW)])

    return k(emb, xplanes, gidx, fgidx)


def kernel(x, f, W1, b1, g1, be1, W2, b2, g2, be2):
    xt3 = jnp.transpose(x, (2, 0, 1))
    xt = jnp.transpose(x, (0, 2, 1))

    s1 = g1 * INV_SQRT
    s2 = g2 * INV_SQRT
    w1s = (W1 * s1[:, None]).T
    b1s = (b1 * s1 + be1)[None, :]
    w2s = (W2 * s2[:, None]).T
    b2s = (b2 * s2 + be2)[None, :]

    emb = _embed(f.reshape(B * N, DIN), w1s, b1s, w2s, b2s)

    xs_t, fps_gidx = _fps(xt3)
    xs = jnp.transpose(xs_t, (1, 2, 0))

    xst = jnp.transpose(xs, (0, 2, 1))
    gidx = jnp.transpose(_ballq(x, xst), (0, 2, 1))

    fg_flat, xg_pl, fs_flat = _sc_gather(
        emb, xt3.reshape(3 * B * N), gidx.reshape(B * S * K),
        fps_gidx.reshape(B * S))

    xg = jnp.transpose(xg_pl.reshape(3, B * S * K), (1, 0)).reshape(B, S, K, 3)
    fg = fg_flat.reshape(B, S, K, DOUT)
    fs = fs_flat.reshape(B, S, DOUT)
    return (xg, fg, xs, fs)


# speedup vs baseline: 18.4278x; 1.0283x over previous
import functools

import jax
import jax.numpy as jnp
import numpy as np
from jax import lax
from jax.experimental import pallas as pl
from jax.experimental.pallas import tpu as pltpu
from jax.experimental.pallas import tpu_sc as plsc

B, N, S, K = 8, 4096, 512, 32
DIN, DMID, DOUT = 128, 128, 256
R2 = np.float32(0.2 ** 2)
INV_SQRT = np.float32(1.0) / np.sqrt(np.float32(1.0 + 1e-5))
NCHUNK = N // 128


def _embed_body(f_ref, w1_ref, b1_ref, w2_ref, b2_ref, o_ref):
    xb = f_ref[...].astype(jnp.bfloat16)
    h = jax.lax.dot_general(xb, w1_ref[...].astype(jnp.bfloat16),
                            (((1,), (0,)), ((), ())),
                            preferred_element_type=jnp.float32)
    h = h + b1_ref[...]
    h = jnp.maximum(h, 0.0)
    h2 = jax.lax.dot_general(h.astype(jnp.bfloat16),
                             w2_ref[...].astype(jnp.bfloat16),
                             (((1,), (0,)), ((), ())),
                             preferred_element_type=jnp.float32)
    o_ref[...] = jnp.maximum(h2 + b2_ref[...], 0.0)


def _embed(f2d, w1s, b1s, w2s, b2s):
    blk = 1024
    return pl.pallas_call(
        _embed_body,
        grid=(B * N // blk,),
        in_specs=[pl.BlockSpec((blk, DIN), lambda i: (i, 0)),
                  pl.BlockSpec((DIN, DMID), lambda i: (0, 0)),
                  pl.BlockSpec((1, DMID), lambda i: (0, 0)),
                  pl.BlockSpec((DMID, DOUT), lambda i: (0, 0)),
                  pl.BlockSpec((1, DOUT), lambda i: (0, 0))],
        out_specs=pl.BlockSpec((blk, DOUT), lambda i: (i, 0)),
        out_shape=jax.ShapeDtypeStruct((B * N, DOUT), jnp.float32),
    )(f2d, w1s, b1s, w2s, b2s)


def _fps_body(xt_ref, xs_ref, gidx_ref, dmin_ref):
    xt = xt_ref[...]
    lane = lax.broadcasted_iota(jnp.int32, (B, N), 1)
    slane = lax.broadcasted_iota(jnp.int32, (B, S), 1)
    slane3 = lax.broadcasted_iota(jnp.int32, (3, B, S), 2)
    boff = lax.broadcasted_iota(jnp.int32, (B, 1), 0) * N
    dmin_ref[...] = jnp.full((B, N), 1e10, jnp.float32)

    def step(i, far):
        gidx_ref[...] = jnp.where(slane == i, far + boff, gidx_ref[...])
        oh = lane == far
        sel = jnp.where(oh[None], xt, 0.0)
        c = jnp.sum(sel, axis=2, keepdims=True)
        xs_ref[...] = jnp.where(slane3 == i, c, xs_ref[...])
        d = xt - c
        d = d * d
        dist = (d[0] + d[1]) + d[2]
        dm = jnp.minimum(dmin_ref[...], dist)
        dmin_ref[...] = dm
        m = jnp.max(dm, axis=1, keepdims=True)
        far_new = jnp.min(jnp.where(dm == m, lane, jnp.int32(N)), axis=1,
                          keepdims=True)
        return far_new

    lax.fori_loop(0, S, step, jnp.zeros((B, 1), jnp.int32))


def _fps(xt):
    return pl.pallas_call(
        _fps_body,
        in_specs=[pl.BlockSpec((3, B, N), lambda: (0, 0, 0))],
        out_specs=[pl.BlockSpec((3, B, S), lambda: (0, 0, 0)),
                   pl.BlockSpec((B, S), lambda: (0, 0))],
        out_shape=[jax.ShapeDtypeStruct((3, B, S), jnp.float32),
                   jax.ShapeDtypeStruct((B, S), jnp.int32)],
        scratch_shapes=[pltpu.VMEM((B, N), jnp.float32)],
    )(xt)


def _ballq_body(xc_ref, at_ref, o_ref, base_scr, acc_scr):
    b = pl.program_id(0)
    j = pl.program_id(1)

    @pl.when(j == 0)
    def _():
        base_scr[...] = jnp.zeros((1, S), jnp.float32)
        acc_scr[...] = jnp.zeros((K, S), jnp.float32)

    xc = xc_ref[0]
    at = at_ref[0]
    eT = jax.lax.dot_general(xc.astype(jnp.bfloat16), at.astype(jnp.bfloat16),
                             (((1,), (0,)), ((), ())),
                             preferred_element_type=jnp.float32)
    s2T = (at[0:1] * at[0:1] + at[1:2] * at[1:2]) + at[2:3] * at[2:3]
    n2T = (xc[:, 0:1] * xc[:, 0:1] + xc[:, 1:2] * xc[:, 1:2]) + xc[:, 2:3] * xc[:, 2:3]
    dT = (eT * (-2.0) + s2T) + n2T
    maskT = jnp.logical_not(dT > R2).astype(jnp.bfloat16)

    tril = (lax.broadcasted_iota(jnp.int32, (128, 128), 1)
            <= lax.broadcasted_iota(jnp.int32, (128, 128), 0)).astype(jnp.bfloat16)
    lc = jax.lax.dot_general(tril, maskT, (((1,), (0,)), ((), ())),
                             preferred_element_type=jnp.float32)
    lc = lc + base_scr[...]
    base_scr[...] = lc[127:128, :]
    u = jnp.minimum(lc, 34.0)

    cols = []
    for k in range(K):
        le = jnp.where(u <= np.float32(k), 1.0, 0.0)
        cols.append(jnp.sum(le, axis=0, keepdims=True))
    acc_scr[...] += jnp.concatenate(cols, axis=0)

    @pl.when(j == NCHUNK - 1)
    def _():
        pos = acc_scr[...]
        pos = jnp.where(pos == np.float32(N), pos[0:1, :], pos)
        o_ref[0] = pos.astype(jnp.int32) + b * N


def _ballq(x, xst):
    return pl.pallas_call(
        _ballq_body,
        grid=(B, NCHUNK),
        in_specs=[pl.BlockSpec((1, 128, 3), lambda b, j: (b, j, 0)),
                  pl.BlockSpec((1, 3, S), lambda b, j: (b, 0, 0))],
        out_specs=pl.BlockSpec((1, K, S), lambda b, j: (b, 0, 0)),
        out_shape=jax.ShapeDtypeStruct((B, K, S), jnp.int32),
        scratch_shapes=[pltpu.VMEM((1, S), jnp.float32),
                        pltpu.VMEM((K, S), jnp.float32)],
    )(x, xst)


try:
    _info = plsc.get_sparse_core_info()
    _NC, _NS = _info.num_cores, _info.num_subcores
except Exception:
    _NC, _NS = 2, 16
_NW = _NC * _NS
_FG_PW = (B * S * K) // _NW
_FS_PW = (B * S) // _NW
_CH = 128
_NIT = _FG_PW // _CH


def _sc_gather(emb, xplanes, gidx, fgidx):
    mesh = plsc.VectorSubcoreMesh(core_axis_name="c", subcore_axis_name="s")

    @functools.partial(
        pl.kernel,
        out_type=[jax.ShapeDtypeStruct((B * S * K, DOUT), jnp.float32),
                  jax.ShapeDtypeStruct((3 * B * S * K,), jnp.float32),
                  jax.ShapeDtypeStruct((B * S, DOUT), jnp.float32)],
        mesh=mesh,
        compiler_params=pltpu.CompilerParams(needs_layout_passes=False),
        scratch_types=[pltpu.VMEM((_FG_PW,), jnp.int32),
                       pltpu.VMEM((_CH, DOUT), jnp.float32),
                       pltpu.VMEM((_CH, DOUT), jnp.float32),
                       pltpu.VMEM((N,), jnp.float32),
                       pltpu.VMEM((N,), jnp.float32),
                       pltpu.VMEM((N,), jnp.float32),
                       pltpu.VMEM((_FG_PW,), jnp.float32),
                       pltpu.VMEM((_FG_PW,), jnp.float32),
                       pltpu.VMEM((_FG_PW,), jnp.float32),
                       pltpu.VMEM((_FS_PW,), jnp.int32),
                       pltpu.SemaphoreType.DMA,
                       pltpu.SemaphoreType.DMA],
    )
    def k(emb_hbm, xpl_hbm, gidx_hbm, fgidx_hbm, fg_hbm, xg_hbm, fs_hbm,
          idx_v, rows_a, rows_b, xt0, xt1, xt2, xo0, xo1, xo2, fidx_v,
          sem_a, sem_b):
        wid = lax.axis_index("s") * _NC + lax.axis_index("c")
        base = wid * _FG_PW
        b = wid // (_NW // B)
        pltpu.sync_copy(gidx_hbm.at[pl.ds(base, _FG_PW)], idx_v)

        xtabs = (xt0, xt1, xt2)
        xouts = (xo0, xo1, xo2)
        for p in range(3):
            pltpu.sync_copy(xpl_hbm.at[pl.ds((p * B + b) * N, N)], xtabs[p])

        def fg_start(j, buf, sem):
            pltpu.async_copy(emb_hbm.at[idx_v.at[pl.ds(j * _CH, _CH)]], buf,
                             sem)

        fg_start(0, rows_a, sem_a)

        def fg_pair(i, _):
            j0 = i * 2
            fg_start(j0 + 1, rows_b, sem_b)
            pltpu.make_async_copy(emb_hbm.at[idx_v.at[pl.ds(0, _CH)]], rows_a,
                                  sem_a).wait()
            pltpu.sync_copy(rows_a, fg_hbm.at[pl.ds(base + j0 * _CH, _CH)])

            @pl.when(i < _NIT // 2 - 1)
            def _():
                fg_start(j0 + 2, rows_a, sem_a)

            pltpu.make_async_copy(emb_hbm.at[idx_v.at[pl.ds(0, _CH)]], rows_b,
                                  sem_b).wait()
            pltpu.sync_copy(rows_b, fg_hbm.at[pl.ds(base + (j0 + 1) * _CH,
                                                    _CH)])
            return 0

        lax.fori_loop(0, _NIT // 2, fg_pair, 0)

        boff = b * N

        def xg_step(i, _):
            ids = idx_v[pl.ds(i * 16, 16)] - boff
            for p in range(3):
                xouts[p][pl.ds(i * 16, 16)] = plsc.load_gather(xtabs[p], [ids])
            return 0

        lax.fori_loop(0, _FG_PW // 16, xg_step, 0)
        for p in range(3):
            pltpu.sync_copy(xouts[p],
                            xg_hbm.at[pl.ds(p * (B * S * K) + base, _FG_PW)])

        fbase = wid * _FS_PW
        pltpu.sync_copy(fgidx_hbm.at[pl.ds(fbase, _FS_PW)], fidx_v)
        pltpu.async_copy(emb_hbm.at[fidx_v], rows_a, sem_a).wait()
        pltpu.sync_copy(rows_a, fs_hbm.at[pl.ds(fbase, _FS_PW)])

    return k(emb, xplanes, gidx, fgidx)


def kernel(x, f, W1, b1, g1, be1, W2, b2, g2, be2):
    xt3 = jnp.transpose(x, (2, 0, 1))
    xt = jnp.transpose(x, (0, 2, 1))

    s1 = g1 * INV_SQRT
    s2 = g2 * INV_SQRT
    w1s = (W1 * s1[:, None]).T
    b1s = (b1 * s1 + be1)[None, :]
    w2s = (W2 * s2[:, None]).T
    b2s = (b2 * s2 + be2)[None, :]

    emb = _embed(f.reshape(B * N, DIN), w1s, b1s, w2s, b2s)

    xs_t, fps_gidx = _fps(xt3)
    xs = jnp.transpose(xs_t, (1, 2, 0))

    xst = jnp.transpose(xs, (0, 2, 1))
    gidx = jnp.transpose(_ballq(x, xst), (0, 2, 1))

    fg_flat, xg_pl, fs_flat = _sc_gather(
        emb, xt3.reshape(3 * B * N), gidx.reshape(B * S * K),
        fps_gidx.reshape(B * S))

    xg = jnp.transpose(xg_pl.reshape(3, B * S * K), (1, 0)).reshape(B, S, K, 3)
    fg = fg_flat.reshape(B, S, K, DOUT)
    fs = fs_flat.reshape(B, S, DOUT)
    return (xg, fg, xs, fs)
